# baseline (speedup 1.0000x reference)
import functools

import jax
import jax.numpy as jnp
from jax import lax
from jax.experimental import pallas as pl
from jax.experimental.pallas import tpu as pltpu
from jax.experimental.pallas import tpu_sc as plsc

NUM_EMOTIONS = 1000
EMBED_DIM = 128
BATCH = 16384

_NC = 2
_NS = 16
_NW = _NC * _NS
_BPW = BATCH // _NW
_CHUNK = 128
_NCHUNK = _BPW // _CHUNK
_SLAB = 64
_LAST_OFF = NUM_EMOTIONS - _SLAB

_mesh = plsc.VectorSubcoreMesh(core_axis_name="c", subcore_axis_name="s")


@functools.partial(
    pl.kernel,
    mesh=_mesh,
    out_type=jax.ShapeDtypeStruct((BATCH, EMBED_DIM), jnp.float32),
    scratch_types=[
        pltpu.VMEM((_NCHUNK, _CHUNK), jnp.int32),
        pltpu.VMEM((_NCHUNK, _CHUNK, EMBED_DIM), jnp.float32),
        pltpu.VMEM_SHARED((NUM_EMOTIONS, EMBED_DIM), jnp.float32),
    ]
    + [pltpu.SemaphoreType.DMA] * _NCHUNK
    + [pltpu.SemaphoreType.DMA, pltpu.SemaphoreType.DMA],
)
def _emb_lookup(ids_hbm, table_hbm, out_hbm, idx_v, rows_v, table_sp, *sems):
    gsems, osem, ssem = sems[:_NCHUNK], sems[_NCHUNK], sems[_NCHUNK + 1]
    sid = lax.axis_index("s")
    wid = sid * _NC + lax.axis_index("c")
    row0 = wid * _BPW
    soff = pl.multiple_of(
        jnp.where(sid < _NS - 1, sid * _SLAB, _LAST_OFF).astype(jnp.int32), 8
    )
    pltpu.async_copy(
        table_hbm.at[pl.ds(soff, _SLAB)], table_sp.at[pl.ds(soff, _SLAB)], ssem
    )
    pltpu.sync_copy(ids_hbm.at[pl.ds(wid * _NCHUNK, _NCHUNK)], idx_v)
    pltpu.make_async_copy(
        table_hbm.at[pl.ds(soff, _SLAB)], table_sp.at[pl.ds(soff, _SLAB)], ssem
    ).wait()
    plsc.subcore_barrier()
    gathers = [
        pltpu.async_copy(table_sp.at[idx_v.at[j]], rows_v.at[j], gsems[j])
        for j in range(_NCHUNK)
    ]
    outs = []
    for j in range(_NCHUNK):
        gathers[j].wait()
        outs.append(
            pltpu.async_copy(
                rows_v.at[j], out_hbm.at[pl.ds(row0 + j * _CHUNK, _CHUNK)], osem
            )
        )
    for o in outs:
        o.wait()


def kernel(emotion_ids, weight):
    ids2d = emotion_ids.astype(jnp.int32).reshape(BATCH // _CHUNK, _CHUNK)
    return _emb_lookup(ids2d, weight)


# speedup vs baseline: 1.0167x; 1.0167x over previous
import functools

import jax
import jax.numpy as jnp
from jax import lax
from jax.experimental import pallas as pl
from jax.experimental.pallas import tpu as pltpu
from jax.experimental.pallas import tpu_sc as plsc

NUM_EMOTIONS = 1000
EMBED_DIM = 128
BATCH = 16384

_NC = 2
_NS = 16
_NW = _NC * _NS
_BPW = BATCH // _NW
_CHUNK = 64
_NCHUNK = _BPW // _CHUNK
_SLAB = 64
_LAST_OFF = NUM_EMOTIONS - _SLAB

_mesh = plsc.VectorSubcoreMesh(core_axis_name="c", subcore_axis_name="s")


@functools.partial(
    pl.kernel,
    mesh=_mesh,
    out_type=jax.ShapeDtypeStruct((BATCH, EMBED_DIM), jnp.float32),
    scratch_types=[
        pltpu.VMEM((_NCHUNK, _CHUNK), jnp.int32),
        pltpu.VMEM((_BPW, EMBED_DIM), jnp.float32),
        pltpu.VMEM_SHARED((NUM_EMOTIONS, EMBED_DIM), jnp.float32),
    ]
    + [pltpu.SemaphoreType.DMA] * _NCHUNK
    + [pltpu.SemaphoreType.DMA, pltpu.SemaphoreType.DMA],
)
def _emb_lookup(ids_hbm, table_hbm, out_hbm, idx_v, rows_v, table_sp, *sems):
    gsems, osem, ssem = sems[:_NCHUNK], sems[_NCHUNK], sems[_NCHUNK + 1]
    sid = lax.axis_index("s")
    wid = sid * _NC + lax.axis_index("c")
    row0 = wid * _BPW
    soff = pl.multiple_of(
        jnp.where(sid < _NS - 1, sid * _SLAB, _LAST_OFF).astype(jnp.int32), 8
    )
    pltpu.async_copy(
        table_hbm.at[pl.ds(soff, _SLAB)], table_sp.at[pl.ds(soff, _SLAB)], ssem
    )
    pltpu.sync_copy(ids_hbm.at[pl.ds(wid * _NCHUNK, _NCHUNK)], idx_v)
    pltpu.make_async_copy(
        table_hbm.at[pl.ds(soff, _SLAB)], table_sp.at[pl.ds(soff, _SLAB)], ssem
    ).wait()
    plsc.subcore_barrier()
    gathers = [
        pltpu.async_copy(
            table_sp.at[idx_v.at[j]],
            rows_v.at[pl.ds(j * _CHUNK, _CHUNK)],
            gsems[j],
        )
        for j in range(_NCHUNK)
    ]
    for j in range(_NCHUNK):
        gathers[j].wait()
        pltpu.async_copy(
            rows_v.at[pl.ds(j * _CHUNK, _CHUNK)],
            out_hbm.at[pl.ds(row0 + j * _CHUNK, _CHUNK)],
            osem,
        )
    pltpu.make_async_copy(rows_v, out_hbm.at[pl.ds(row0, _BPW)], osem).wait()


def kernel(emotion_ids, weight):
    ids2d = emotion_ids.astype(jnp.int32).reshape(BATCH // _CHUNK, _CHUNK)
    return _emb_lookup(ids2d, weight)
